# baseline (speedup 1.0000x reference)
import functools

import jax
import jax.numpy as jnp
from jax import lax
from jax.experimental import pallas as pl
from jax.experimental.pallas import tpu as pltpu
from jax.experimental.pallas import tpu_sc as plsc

N, C, H, W = 4, 96, 224, 224
B = N * H * W
HW = H * W
NC, NS, L = 2, 16, 16
NW = NC * NS
BPT = B // NW
K = 112
NCHUNK = BPT // K
ROWS_PT = BPT // W


def _grid_kernel(table, gx_hbm, gy_hbm, out_hbm,
                 gx_v, gy_v, idx_v, w_v, rows_v, out_v,
                 gsem0, gsem1, ssem0, ssem1):
    wid = lax.axis_index("s") * NC + lax.axis_index("c")
    base = wid * BPT
    n_img = wid // (HW // BPT)
    n_base = n_img * HW
    h_start = (wid % (HW // BPT)) * ROWS_PT

    pltpu.sync_copy(gx_hbm.at[pl.ds(base, BPT)], gx_v)
    pltpu.sync_copy(gy_hbm.at[pl.ds(base, BPT)], gy_v)

    lane = lax.iota(jnp.int32, L)
    chvec = [lane + cg * L for cg in range(C // L)]

    def prime(cidx, slot):
        for i in range(K // L):
            s = pl.ds(i * L, L)
            gx = gx_v[pl.ds(cidx * K + i * L, L)]
            gy = gy_v[pl.ds(cidx * K + i * L, L)]
            ix = (gx + 1.0) * 0.5 * (W - 1)
            iy = (gy + 1.0) * 0.5 * (H - 1)
            ix0 = ix.astype(jnp.int32)
            iy0 = iy.astype(jnp.int32)
            wx1 = ix - ix0.astype(jnp.float32)
            wy1 = iy - iy0.astype(jnp.float32)
            wx0 = 1.0 - wx1
            wy0 = 1.0 - wy1
            ix1 = jnp.minimum(ix0 + 1, W - 1)
            iy1 = jnp.minimum(iy0 + 1, H - 1)
            r0 = n_base + iy0 * W
            r1 = n_base + iy1 * W
            idx_v[slot, 0, s] = r0 + ix0
            idx_v[slot, 1, s] = r0 + ix1
            idx_v[slot, 2, s] = r1 + ix0
            idx_v[slot, 3, s] = r1 + ix1
            w_v[slot, 0, s] = wy0 * wx0
            w_v[slot, 1, s] = wy0 * wx1
            w_v[slot, 2, s] = wy1 * wx0
            w_v[slot, 3, s] = wy1 * wx1

    def fire(cidx, slot, sem):
        prime(cidx, slot)
        for c in range(4):
            pltpu.make_async_copy(table.at[idx_v.at[slot, c]],
                                  rows_v.at[slot, c], sem).start()

    def drain_gather(slot, sem):
        for c in range(4):
            pltpu.make_async_copy(table.at[idx_v.at[slot, c]],
                                  rows_v.at[slot, c], sem).wait()

    def _scatter_dst(cidx):
        h = h_start + cidx // (W // K)
        w0 = (cidx % (W // K)) * K
        return out_hbm.at[n_img, :, h, pl.ds(w0, K)]

    def fire_scatter(cidx, slot, sem):
        pltpu.make_async_copy(out_v.at[slot], _scatter_dst(cidx), sem).start()

    def drain_scatter(cidx, slot, sem):
        pltpu.make_async_copy(out_v.at[slot], _scatter_dst(cidx), sem).wait()

    def blend(slot):
        out_flat = out_v.at[slot]

        def gbody(gi, _):
            wv = [w_v[slot, c, pl.ds(gi * L, L)] for c in range(4)]
            for jj in range(L):
                j = gi * L + jj
                w00 = jnp.full((L,), wv[0][jj], jnp.float32)
                w01 = jnp.full((L,), wv[1][jj], jnp.float32)
                w10 = jnp.full((L,), wv[2][jj], jnp.float32)
                w11 = jnp.full((L,), wv[3][jj], jnp.float32)
                jvec = jnp.full((L,), j, jnp.int32)
                for cg in range(C // L):
                    cs = pl.ds(cg * L, L)
                    acc = (w00 * rows_v[slot, 0, j, cs]
                           + w01 * rows_v[slot, 1, j, cs]
                           + w10 * rows_v[slot, 2, j, cs]
                           + w11 * rows_v[slot, 3, j, cs])
                    plsc.store_scatter(out_flat, [chvec[cg], jvec], acc)
            return 0

        lax.fori_loop(0, K // L, gbody, 0)

    fire(0, 0, gsem0)
    fire(1, 1, gsem1)

    def step_body(step, _):
        for b, gs, ss in ((0, gsem0, ssem0), (1, gsem1, ssem1)):
            c = step * 2 + b
            drain_gather(b, gs)

            @pl.when(step >= 1)
            def _():
                drain_scatter(c - 2, b, ss)

            blend(b)
            fire_scatter(c, b, ss)

            @pl.when(step < NCHUNK // 2 - 1)
            def _():
                fire(c + 2, b, gs)
        return 0

    lax.fori_loop(0, NCHUNK // 2, step_body, 0)
    drain_scatter(NCHUNK - 2, 0, ssem0)
    drain_scatter(NCHUNK - 1, 1, ssem1)


_grid_call = functools.partial(
    pl.kernel,
    out_type=jax.ShapeDtypeStruct((N, C, H, W), jnp.float32),
    mesh=plsc.VectorSubcoreMesh(core_axis_name="c", subcore_axis_name="s"),
    scratch_types=[
        pltpu.VMEM((BPT,), jnp.float32),
        pltpu.VMEM((BPT,), jnp.float32),
        pltpu.VMEM((2, 4, K), jnp.int32),
        pltpu.VMEM((2, 4, K), jnp.float32),
        pltpu.VMEM((2, 4, K, C), jnp.float32),
        pltpu.VMEM((2, C, K), jnp.float32),
        pltpu.SemaphoreType.DMA,
        pltpu.SemaphoreType.DMA,
        pltpu.SemaphoreType.DMA,
        pltpu.SemaphoreType.DMA,
    ],
    compiler_params=pltpu.CompilerParams(use_tc_tiling_on_sc=False,
                                         needs_layout_passes=False),
)(_grid_kernel)


def kernel(x, g, e):
    del e
    table = x.transpose(0, 2, 3, 1).reshape(B, C)
    gflat = g.reshape(B, 2)
    return _grid_call(table, gflat[:, 0], gflat[:, 1])


# speedup vs baseline: 1.1035x; 1.1035x over previous
import functools

import jax
import jax.numpy as jnp
from jax import lax
from jax.experimental import pallas as pl
from jax.experimental.pallas import tpu as pltpu
from jax.experimental.pallas import tpu_sc as plsc

N, C, H, W = 4, 96, 224, 224
B = N * H * W
HW = H * W
NC, NS, L = 2, 16, 16
NW = NC * NS
BPT = B // NW
K = 112
NCHUNK = BPT // K
ROWS_PT = BPT // W


def _grid_kernel(table, gx_hbm, gy_hbm, out_hbm,
                 gx_v, gy_v, idx_v, w_v, rows_v, out_v,
                 gsem0, gsem1, ssem0, ssem1):
    wid = lax.axis_index("s") * NC + lax.axis_index("c")
    base = wid * BPT
    n_img = wid // (HW // BPT)
    n_base = n_img * HW
    h_start = (wid % (HW // BPT)) * ROWS_PT

    pltpu.sync_copy(gx_hbm.at[pl.ds(base, BPT)], gx_v)
    pltpu.sync_copy(gy_hbm.at[pl.ds(base, BPT)], gy_v)

    lane = lax.iota(jnp.int32, L)
    chvec = [lane + cg * L for cg in range(C // L)]

    def prime(cidx, slot):
        for i in range(K // L):
            s = pl.ds(i * L, L)
            gx = gx_v[pl.ds(cidx * K + i * L, L)]
            gy = gy_v[pl.ds(cidx * K + i * L, L)]
            ix = (gx + 1.0) * 0.5 * (W - 1)
            iy = (gy + 1.0) * 0.5 * (H - 1)
            ix0 = ix.astype(jnp.int32)
            iy0 = iy.astype(jnp.int32)
            wx1 = ix - ix0.astype(jnp.float32)
            wy1 = iy - iy0.astype(jnp.float32)
            wx0 = 1.0 - wx1
            wy0 = 1.0 - wy1
            ix1 = jnp.minimum(ix0 + 1, W - 1)
            iy1 = jnp.minimum(iy0 + 1, H - 1)
            r0 = n_base + iy0 * W
            r1 = n_base + iy1 * W
            idx_v[slot, 0, s] = r0 + ix0
            idx_v[slot, 1, s] = r0 + ix1
            idx_v[slot, 2, s] = r1 + ix0
            idx_v[slot, 3, s] = r1 + ix1
            w_v[slot, 0, s] = wy0 * wx0
            w_v[slot, 1, s] = wy0 * wx1
            w_v[slot, 2, s] = wy1 * wx0
            w_v[slot, 3, s] = wy1 * wx1

    def fire(cidx, slot, sem):
        prime(cidx, slot)
        for c in range(4):
            pltpu.make_async_copy(table.at[idx_v.at[slot, c]],
                                  rows_v.at[slot, c], sem).start()

    def drain_gather(slot, sem):
        for c in range(4):
            pltpu.make_async_copy(table.at[idx_v.at[slot, c]],
                                  rows_v.at[slot, c], sem).wait()

    def _scatter_dst(cidx):
        h = h_start + cidx // (W // K)
        w0 = (cidx % (W // K)) * K
        return out_hbm.at[n_img, :, h, pl.ds(w0, K)]

    def fire_scatter(cidx, slot, sem):
        pltpu.make_async_copy(out_v.at[slot], _scatter_dst(cidx), sem).start()

    def drain_scatter(cidx, slot, sem):
        pltpu.make_async_copy(out_v.at[slot], _scatter_dst(cidx), sem).wait()

    def blend(slot):
        out_flat = out_v.at[slot]

        def gbody(gi, _):
            wv = [w_v[slot, c, pl.ds(gi * L, L)] for c in range(4)]
            for jj in range(L):
                j = gi * L + jj
                w00 = jnp.full((L,), wv[0][jj], jnp.float32)
                w01 = jnp.full((L,), wv[1][jj], jnp.float32)
                w10 = jnp.full((L,), wv[2][jj], jnp.float32)
                w11 = jnp.full((L,), wv[3][jj], jnp.float32)
                jvec = jnp.full((L,), j, jnp.int32)
                for cg in range(C // L):
                    cs = pl.ds(cg * L, L)
                    acc = (w00 * rows_v[slot, 0, j, cs]
                           + w01 * rows_v[slot, 1, j, cs]
                           + w10 * rows_v[slot, 2, j, cs]
                           + w11 * rows_v[slot, 3, j, cs])
                    plsc.store_scatter(out_flat, [chvec[cg], jvec], acc)
            return 0

        lax.fori_loop(0, K // L, gbody, 0)

    fire(0, 0, gsem0)
    fire(1, 1, gsem1)

    def step_body(step, _):
        for b, gs, ss in ((0, gsem0, ssem0), (1, gsem1, ssem1)):
            c = step * 2 + b
            drain_gather(b, gs)

            @pl.when(step >= 1)
            def _():
                drain_scatter(c - 2, b, ss)

            blend(b)
            fire_scatter(c, b, ss)

            @pl.when(step < NCHUNK // 2 - 1)
            def _():
                fire(c + 2, b, gs)
        return 0

    lax.fori_loop(0, NCHUNK // 2, step_body, 0)
    drain_scatter(NCHUNK - 2, 0, ssem0)
    drain_scatter(NCHUNK - 1, 1, ssem1)


_grid_call = functools.partial(
    pl.kernel,
    out_type=jax.ShapeDtypeStruct((N, C, H, W), jnp.float32),
    mesh=plsc.VectorSubcoreMesh(core_axis_name="c", subcore_axis_name="s"),
    scratch_types=[
        pltpu.VMEM((BPT,), jnp.float32),
        pltpu.VMEM((BPT,), jnp.float32),
        pltpu.VMEM((2, 4, K), jnp.int32),
        pltpu.VMEM((2, 4, K), jnp.float32),
        pltpu.VMEM((2, 4, K, C), jnp.float32),
        pltpu.VMEM((2, C, K), jnp.float32),
        pltpu.SemaphoreType.DMA,
        pltpu.SemaphoreType.DMA,
        pltpu.SemaphoreType.DMA,
        pltpu.SemaphoreType.DMA,
    ],
    compiler_params=pltpu.CompilerParams(use_tc_tiling_on_sc=False,
                                         needs_layout_passes=False),
)(_grid_kernel)


def _tpose_body(x_ref, o_ref):
    o_ref[0] = x_ref[0].T


_TW = 4 * W


_tpose_call = pl.pallas_call(
    _tpose_body,
    grid=(N, HW // _TW),
    in_specs=[pl.BlockSpec((1, C, _TW), lambda n, h: (n, 0, h))],
    out_specs=pl.BlockSpec((1, _TW, C), lambda n, h: (n, h, 0)),
    out_shape=jax.ShapeDtypeStruct((N, HW, C), jnp.float32),
)


def kernel(x, g, e):
    del e
    table = _tpose_call(x.reshape(N, C, HW)).reshape(B, C)
    gflat = g.reshape(B, 2)
    return _grid_call(table, gflat[:, 0], gflat[:, 1])


# speedup vs baseline: 1.1397x; 1.0328x over previous
import functools

import jax
import jax.numpy as jnp
from jax import lax
from jax.experimental import pallas as pl
from jax.experimental.pallas import tpu as pltpu
from jax.experimental.pallas import tpu_sc as plsc

N, C, H, W = 4, 96, 224, 224
B = N * H * W
HW = H * W
NC, NS, L = 2, 16, 16
NW = NC * NS
BPT = B // NW
K = 112
NCHUNK = BPT // K
ROWS_PT = BPT // W


def _grid_kernel(table, gxy_hbm, out_hbm,
                 gxy_v, idx_v, w_v, rows_v, out_v,
                 gsem0, gsem1, ssem0, ssem1):
    wid = lax.axis_index("s") * NC + lax.axis_index("c")
    base = wid * BPT
    n_img = wid // (HW // BPT)
    n_base = n_img * HW
    h_start = (wid % (HW // BPT)) * ROWS_PT

    pltpu.sync_copy(gxy_hbm.at[pl.ds(2 * base, 2 * BPT)], gxy_v)

    lane = lax.iota(jnp.int32, L)
    chvec = [lane + cg * L for cg in range(C // L)]

    def prime(cidx, slot):
        for i in range(K // L):
            s = pl.ds(i * L, L)
            pvec = 2 * (cidx * K + i * L + lane)
            gx = plsc.load_gather(gxy_v, [pvec])
            gy = plsc.load_gather(gxy_v, [pvec + 1])
            ix = (gx + 1.0) * 0.5 * (W - 1)
            iy = (gy + 1.0) * 0.5 * (H - 1)
            ix0 = ix.astype(jnp.int32)
            iy0 = iy.astype(jnp.int32)
            wx1 = ix - ix0.astype(jnp.float32)
            wy1 = iy - iy0.astype(jnp.float32)
            wx0 = 1.0 - wx1
            wy0 = 1.0 - wy1
            ix1 = jnp.minimum(ix0 + 1, W - 1)
            iy1 = jnp.minimum(iy0 + 1, H - 1)
            r0 = n_base + iy0 * W
            r1 = n_base + iy1 * W
            idx_v[slot, 0, s] = r0 + ix0
            idx_v[slot, 1, s] = r0 + ix1
            idx_v[slot, 2, s] = r1 + ix0
            idx_v[slot, 3, s] = r1 + ix1
            w_v[slot, 0, s] = wy0 * wx0
            w_v[slot, 1, s] = wy0 * wx1
            w_v[slot, 2, s] = wy1 * wx0
            w_v[slot, 3, s] = wy1 * wx1

    def fire(cidx, slot, sem):
        prime(cidx, slot)
        for c in range(4):
            pltpu.make_async_copy(table.at[idx_v.at[slot, c]],
                                  rows_v.at[slot, c], sem).start()

    def drain_gather(slot, sem):
        for c in range(4):
            pltpu.make_async_copy(table.at[idx_v.at[slot, c]],
                                  rows_v.at[slot, c], sem).wait()

    def _scatter_dst(cidx):
        h = h_start + cidx // (W // K)
        w0 = (cidx % (W // K)) * K
        return out_hbm.at[n_img, :, h, pl.ds(w0, K)]

    def fire_scatter(cidx, slot, sem):
        pltpu.make_async_copy(out_v.at[slot], _scatter_dst(cidx), sem).start()

    def drain_scatter(cidx, slot, sem):
        pltpu.make_async_copy(out_v.at[slot], _scatter_dst(cidx), sem).wait()

    def blend(slot):
        out_flat = out_v.at[slot]

        def gbody(gi, _):
            wv = [w_v[slot, c, pl.ds(gi * L, L)] for c in range(4)]
            for jj in range(L):
                j = gi * L + jj
                w00 = jnp.full((L,), wv[0][jj], jnp.float32)
                w01 = jnp.full((L,), wv[1][jj], jnp.float32)
                w10 = jnp.full((L,), wv[2][jj], jnp.float32)
                w11 = jnp.full((L,), wv[3][jj], jnp.float32)
                jvec = jnp.full((L,), j, jnp.int32)
                for cg in range(C // L):
                    cs = pl.ds(cg * L, L)
                    acc = (w00 * rows_v[slot, 0, j, cs]
                           + w01 * rows_v[slot, 1, j, cs]
                           + w10 * rows_v[slot, 2, j, cs]
                           + w11 * rows_v[slot, 3, j, cs])
                    plsc.store_scatter(out_flat, [chvec[cg], jvec], acc)
            return 0

        lax.fori_loop(0, K // L, gbody, 0)

    fire(0, 0, gsem0)
    fire(1, 1, gsem1)

    def step_body(step, _):
        for b, gs, ss in ((0, gsem0, ssem0), (1, gsem1, ssem1)):
            c = step * 2 + b
            drain_gather(b, gs)

            @pl.when(step >= 1)
            def _():
                drain_scatter(c - 2, b, ss)

            blend(b)
            fire_scatter(c, b, ss)

            @pl.when(step < NCHUNK // 2 - 1)
            def _():
                fire(c + 2, b, gs)
        return 0

    lax.fori_loop(0, NCHUNK // 2, step_body, 0)
    drain_scatter(NCHUNK - 2, 0, ssem0)
    drain_scatter(NCHUNK - 1, 1, ssem1)


_grid_call = functools.partial(
    pl.kernel,
    out_type=jax.ShapeDtypeStruct((N, C, H, W), jnp.float32),
    mesh=plsc.VectorSubcoreMesh(core_axis_name="c", subcore_axis_name="s"),
    scratch_types=[
        pltpu.VMEM((2 * BPT,), jnp.float32),
        pltpu.VMEM((2, 4, K), jnp.int32),
        pltpu.VMEM((2, 4, K), jnp.float32),
        pltpu.VMEM((2, 4, K, C), jnp.float32),
        pltpu.VMEM((2, C, K), jnp.float32),
        pltpu.SemaphoreType.DMA,
        pltpu.SemaphoreType.DMA,
        pltpu.SemaphoreType.DMA,
        pltpu.SemaphoreType.DMA,
    ],
    compiler_params=pltpu.CompilerParams(use_tc_tiling_on_sc=False,
                                         needs_layout_passes=False),
)(_grid_kernel)


_HB = 8


def _tpose_body(x_ref, o_ref):
    for r in range(_HB):
        o_ref[0, r] = x_ref[0, :, r, :].T


_tpose_call = pl.pallas_call(
    _tpose_body,
    grid=(N, H // _HB),
    in_specs=[pl.BlockSpec((1, C, _HB, W), lambda n, h: (n, 0, h, 0))],
    out_specs=pl.BlockSpec((1, _HB, W, C), lambda n, h: (n, h, 0, 0)),
    out_shape=jax.ShapeDtypeStruct((N, H, W, C), jnp.float32),
)


def kernel(x, g, e):
    del e
    table = _tpose_call(x).reshape(B, C)
    return _grid_call(table, g.reshape(2 * B))
